# baseline (speedup 1.0000x reference)
import functools

import jax
import jax.numpy as jnp
from jax import lax
from jax.experimental import pallas as pl
from jax.experimental.pallas import tpu as pltpu
from jax.experimental.pallas import tpu_sc as plsc

_TL = 0.6
_L = 2.0

_B = 4
_HW = 576
_C = 384
_NC = 2
_NS = 16
_LANES = 16
_NW = _NC * _NS
_NGRP = _B * _HW // _LANES
_GRP_PER_W = -(-_NGRP // _NW)
_CBLK = _HW // _LANES


def _dist_body(x_ref, d_ref):
    fmp = x_ref[0]
    fm = fmp[:_HW]
    sq = jnp.sum(fm * fm, axis=1, keepdims=True)
    sqp = jnp.sum(fmp * fmp, axis=1, keepdims=True)
    gram = lax.dot_general(
        fm, fmp,
        dimension_numbers=(((1,), (1,)), ((), ())),
        preferred_element_type=jnp.float32,
    )
    d2 = sq + sqp.T - 2.0 * gram
    d_ref[0] = jnp.sqrt(jnp.maximum(d2, 1e-12))


_sc_mesh = plsc.VectorSubcoreMesh(
    core_axis_name="c", subcore_axis_name="s",
    num_cores=_NC, num_subcores=_NS,
)


def _insert3(m1, m2, m3, v):
    t1 = jnp.minimum(m1, v)
    h1 = jnp.maximum(m1, v)
    t2 = jnp.minimum(m2, h1)
    h2 = jnp.maximum(m2, h1)
    t3 = jnp.minimum(m3, h2)
    return t1, t2, t3


@functools.partial(
    pl.kernel,
    out_type=jax.ShapeDtypeStruct((_NGRP, _LANES), jnp.float32),
    mesh=_sc_mesh,
    compiler_params=pltpu.CompilerParams(needs_layout_passes=False),
    scratch_types=[
        pltpu.VMEM((2, _LANES, _HW + 1), jnp.float32),
        pltpu.VMEM((_GRP_PER_W, _LANES), jnp.float32),
        pltpu.SemaphoreType.DMA,
        pltpu.SemaphoreType.DMA,
        pltpu.SemaphoreType.DMA,
    ],
)
def _sc_select(d_hbm, out_hbm, slabs, obuf, sem0, sem1, osem):
    wid = lax.axis_index("s") * _NC + lax.axis_index("c")
    ridx = lax.iota(jnp.int32, _LANES)
    inf = jnp.full((_LANES,), jnp.inf, jnp.float32)
    zero = jnp.zeros((_LANES,), jnp.float32)
    izero = jnp.zeros((_LANES,), jnp.int32)
    sems = (sem0, sem1)

    def group_of(k):
        if k == _GRP_PER_W - 1:
            return _NGRP - _NW + wid
        return wid + _NW * k

    in_flight = [pltpu.async_copy(d_hbm.at[group_of(0)], slabs.at[0], sem0)]
    out_handles = []

    for k in range(_GRP_PER_W):
        g = group_of(k)
        in_flight[0].wait()
        if k + 1 < _GRP_PER_W:
            in_flight[0] = pltpu.async_copy(
                d_hbm.at[group_of(k + 1)], slabs.at[(k + 1) % 2],
                sems[(k + 1) % 2])

        slab = slabs.at[k % 2]

        @plsc.parallel_loop(
            0, _CBLK, carry=(inf, inf, inf, zero, inf, inf, inf, zero))
        def body(t, carry):
            m1a, m2a, m3a, mxa, m1b, m2b, m3b, mxb = carry
            cbase = izero + t * _LANES
            for u in range(0, _LANES, 2):
                va = plsc.load_gather(slab, [ridx, cbase + u])
                vb = plsc.load_gather(slab, [ridx, cbase + (u + 1)])
                m1a, m2a, m3a = _insert3(m1a, m2a, m3a, va)
                m1b, m2b, m3b = _insert3(m1b, m2b, m3b, vb)
                mxa = jnp.maximum(mxa, va)
                mxb = jnp.maximum(mxb, vb)
            return m1a, m2a, m3a, mxa, m1b, m2b, m3b, mxb

        m1, m2, m3, mx, m1b, m2b, m3b, mxb = body
        m1, m2, m3 = _insert3(m1, m2, m3, m1b)
        m1, m2, m3 = _insert3(m1, m2, m3, m2b)
        m1, m2, m3 = _insert3(m1, m2, m3, m3b)
        mx = jnp.maximum(mx, mxb)

        pred = jnp.where(
            m2 / m3 < _TL,
            2.0 / (1.0 + jnp.exp(m2)),
            2.0 / (1.0 + _L * jnp.exp(mx)),
        )
        obuf[k, :] = pred
        out_handles.append(
            pltpu.async_copy(obuf.at[k], out_hbm.at[g], osem))

    for h in out_handles:
        h.wait()


def kernel(x):
    b, h, w, c = x.shape
    hw = h * w
    hw1 = hw + 1
    fmp = jnp.pad(x.reshape(b, hw, c), ((0, 0), (0, 1), (0, 0)))
    d = pl.pallas_call(
        _dist_body,
        grid=(b,),
        in_specs=[pl.BlockSpec((1, hw1, c), lambda i: (i, 0, 0))],
        out_specs=pl.BlockSpec((1, hw, hw1), lambda i: (i, 0, 0)),
        out_shape=jax.ShapeDtypeStruct((b, hw, hw1), jnp.float32),
    )(fmp)
    pred = _sc_select(d.reshape(_NGRP, _LANES, hw1))
    return pred.reshape(b, h, w)


# speedup vs baseline: 1.5566x; 1.5566x over previous
import functools

import jax
import jax.numpy as jnp
from jax import lax
from jax.experimental import pallas as pl
from jax.experimental.pallas import tpu as pltpu
from jax.experimental.pallas import tpu_sc as plsc

_TL = 0.6
_L = 2.0

_B = 4
_HW = 576
_C = 384
_NC = 2
_NS = 16
_LANES = 16
_NW = _NC * _NS
_NGRP = _B * _HW // _LANES
_GRP_PER_W = -(-_NGRP // _NW)
_CBLK = _HW // _LANES


def _dist_body(x_ref, d_ref):
    fm = x_ref[0]
    sq = jnp.sum(fm * fm, axis=1, keepdims=True)
    gram = lax.dot_general(
        fm, fm,
        dimension_numbers=(((1,), (1,)), ((), ())),
        preferred_element_type=jnp.float32,
    )
    d2 = sq + sq.T - 2.0 * gram
    d_ref[0] = jnp.sqrt(jnp.maximum(d2, 1e-12))


_sc_mesh = plsc.VectorSubcoreMesh(
    core_axis_name="c", subcore_axis_name="s",
    num_cores=_NC, num_subcores=_NS,
)


def _insert3(m1, m2, m3, v):
    t1 = jnp.minimum(m1, v)
    h1 = jnp.maximum(m1, v)
    t2 = jnp.minimum(m2, h1)
    h2 = jnp.maximum(m2, h1)
    t3 = jnp.minimum(m3, h2)
    return t1, t2, t3


@functools.partial(
    pl.kernel,
    out_type=jax.ShapeDtypeStruct((_NGRP, _LANES), jnp.float32),
    mesh=_sc_mesh,
    compiler_params=pltpu.CompilerParams(needs_layout_passes=False),
    scratch_types=[
        pltpu.VMEM((2, _LANES, _HW), jnp.float32),
        pltpu.VMEM((_GRP_PER_W, _LANES), jnp.float32),
        pltpu.SemaphoreType.DMA,
        pltpu.SemaphoreType.DMA,
        pltpu.SemaphoreType.DMA,
    ],
)
def _sc_select(d_hbm, out_hbm, slabs, obuf, sem0, sem1, osem):
    wid = lax.axis_index("s") * _NC + lax.axis_index("c")
    iot = lax.iota(jnp.int32, _LANES)
    inf = jnp.full((_LANES,), jnp.inf, jnp.float32)
    zero = jnp.zeros((_LANES,), jnp.float32)
    rot_idx = [(iot + sh) & (_LANES - 1) for sh in (8, 4, 2, 1)]
    sems = (sem0, sem1)

    def group_of(k):
        if k == _GRP_PER_W - 1:
            return _NGRP - _NW + wid
        return wid + _NW * k

    def _rot(v, idx):
        return v.at[idx].get(mode="promise_in_bounds")

    in_flight = [pltpu.async_copy(d_hbm.at[group_of(0)], slabs.at[0], sem0)]
    out_handles = []

    for k in range(_GRP_PER_W):
        g = group_of(k)
        in_flight[0].wait()
        if k + 1 < _GRP_PER_W:
            in_flight[0] = pltpu.async_copy(
                d_hbm.at[group_of(k + 1)], slabs.at[(k + 1) % 2],
                sems[(k + 1) % 2])

        slab = slabs.at[k % 2]

        def row_body(r, resvec):
            m1, m2, m3, mx = inf, inf, inf, zero
            for t in range(_CBLK):
                v = slab[r, pl.ds(t * _LANES, _LANES)]
                m1, m2, m3 = _insert3(m1, m2, m3, v)
                mx = jnp.maximum(mx, v)
            for idx in rot_idx:
                r1, r2, r3 = _rot(m1, idx), _rot(m2, idx), _rot(m3, idx)
                rx = _rot(mx, idx)
                m1, m2, m3 = _insert3(m1, m2, m3, r1)
                m1, m2, m3 = _insert3(m1, m2, m3, r2)
                m1, m2, m3 = _insert3(m1, m2, m3, r3)
                mx = jnp.maximum(mx, rx)
            pred = jnp.where(
                m2 / m3 < _TL,
                2.0 / (1.0 + jnp.exp(m2)),
                2.0 / (1.0 + _L * jnp.exp(mx)),
            )
            return jnp.where(iot == r, pred, resvec)

        res = lax.fori_loop(0, _LANES, row_body, zero)
        obuf[k, :] = res
        out_handles.append(
            pltpu.async_copy(obuf.at[k], out_hbm.at[g], osem))

    for h in out_handles:
        h.wait()


def kernel(x):
    b, h, w, c = x.shape
    hw = h * w
    fm = x.reshape(b, hw, c)
    d = pl.pallas_call(
        _dist_body,
        grid=(b,),
        in_specs=[pl.BlockSpec((1, hw, c), lambda i: (i, 0, 0))],
        out_specs=pl.BlockSpec((1, hw, hw), lambda i: (i, 0, 0)),
        out_shape=jax.ShapeDtypeStruct((b, hw, hw), jnp.float32),
    )(fm)
    pred = _sc_select(d.reshape(_NGRP, _LANES, hw))
    return pred.reshape(b, h, w)
